# baseline (speedup 1.0000x reference)
import jax
import jax.numpy as jnp
from jax import lax
from jax.experimental import pallas as pl
from jax.experimental.pallas import tpu as pltpu
from jax.experimental.pallas import tpu_sc as plsc

_B, _N, _D = 8, 1024, 64
_K = 8192
_HEADS = 4
_QK_DIM = 64

_TT = 1024
_KT = 1024
_NT = (_B * _N) // _TT
_NK = _K // _KT

_NC, _NS = 2, 16
_NW = _NC * _NS
_CPW = _K // _NW
_TCHUNK = 256
_NTCH = (_B * _N) // _TCHUNK
_GRP = 16
_CW = 16


def _l2norm_rows(x):
    n = jnp.sqrt(jnp.sum(x * x, axis=-1, keepdims=True))
    return x / jnp.maximum(n, 1e-12)


def _norm_body(x_ref, m_ref, xn_ref, mn_ref):
    xn_ref[...] = _l2norm_rows(x_ref[...])
    mn_ref[...] = _l2norm_rows(m_ref[...])


def _assign_body(xn_ref, mn_ref, bkt_ref, best_ref, bidx_ref):
    j = pl.program_id(1)
    xn = xn_ref[...]
    mn = mn_ref[...]
    dist = lax.dot_general(
        mn, xn, (((1,), (1,)), ((), ())),
        preferred_element_type=jnp.float32,
        precision=lax.Precision.DEFAULT)
    maxv = jnp.max(dist, axis=0)
    gidx = lax.broadcasted_iota(jnp.int32, dist.shape, 0) + j * _KT
    cand = jnp.min(jnp.where(dist == maxv[None, :], gidx, jnp.int32(_K)),
                   axis=0)

    @pl.when(j == 0)
    def _():
        best_ref[0, :] = maxv
        bidx_ref[0, :] = cand

    @pl.when(j > 0)
    def _():
        bv = best_ref[0, :]
        upd = maxv > bv
        best_ref[0, :] = jnp.where(upd, maxv, bv)
        bidx_ref[0, :] = jnp.where(upd, cand, bidx_ref[0, :])

    @pl.when(j == _NK - 1)
    def _():
        bkt_ref[0, 0, :] = bidx_ref[0, :]


def _sc_scatter_body(xn_hbm, bkt_hbm, sums_hbm, cnt_hbm,
                     x_v, idx_v, table_v, cnt_v):
    cid = lax.axis_index("c")
    sid = lax.axis_index("s")
    wid = cid * _NS + sid
    cbase = wid * _CPW
    z16 = jnp.zeros((_GRP,), jnp.float32)
    ones16 = jnp.ones((_GRP,), jnp.float32)

    def _zero_row(i, carry):
        for r in range(_D // _GRP):
            table_v[i, pl.ds(r * _GRP, _GRP)] = z16
        cnt_v[i, pl.ds(0, _GRP)] = z16
        return carry

    lax.fori_loop(0, _CPW, _zero_row, 0)

    for c in range(_NTCH):
        pltpu.sync_copy(xn_hbm.at[pl.ds(c * _TCHUNK, _TCHUNK)], x_v)
        pltpu.sync_copy(bkt_hbm.at[pl.ds(c * _TCHUNK, _TCHUNK)], idx_v)

        def _group(g, carry):
            bv = idx_v[pl.ds(g * _GRP, _GRP)]
            for j in range(_GRP):
                off = bv[j] - cbase

                @pl.when((off >= 0) & (off < _CPW))
                def _(off=off, j=j, g=g):
                    tok = g * _GRP + j
                    for r in range(_D // _GRP):
                        xv = x_v[tok, pl.ds(r * _GRP, _GRP)]
                        plsc.addupdate(
                            table_v.at[off, pl.ds(r * _GRP, _GRP)], xv)
                    plsc.addupdate(cnt_v.at[off, pl.ds(0, _GRP)], ones16)

            return carry

        lax.fori_loop(0, _TCHUNK // _GRP, _group, 0)

    pltpu.sync_copy(table_v, sums_hbm.at[pl.ds(wid * _CPW, _CPW)])
    pltpu.sync_copy(cnt_v, cnt_hbm.at[pl.ds(wid * _CPW, _CPW)])


def _finish_body(sums_ref, cnt_ref, mn_ref, wk_ref, wv_ref,
                 xg_ref, k_ref, v_ref):
    sums = sums_ref[...]
    cnt = cnt_ref[:, 0:1]
    mn = mn_ref[...]
    xg = jnp.where(cnt == 0.0, mn, _l2norm_rows(sums))
    xg_ref[...] = xg
    k_ref[...] = lax.dot_general(
        xg, wk_ref[...], (((1,), (1,)), ((), ())),
        preferred_element_type=jnp.float32,
        precision=lax.Precision.HIGHEST)
    v_ref[...] = lax.dot_general(
        xg, wv_ref[...], (((1,), (1,)), ((), ())),
        preferred_element_type=jnp.float32,
        precision=lax.Precision.HIGHEST)


def kernel(normed_x, x_means, W_k, W_v):
    x = normed_x.reshape(_B * _N, _D)

    xn, mn = pl.pallas_call(
        _norm_body,
        grid=(_NT,),
        in_specs=[
            pl.BlockSpec((_TT, _D), lambda i: (i, 0)),
            pl.BlockSpec((_KT, _D), lambda i: (i, 0)),
        ],
        out_specs=[
            pl.BlockSpec((_TT, _D), lambda i: (i, 0)),
            pl.BlockSpec((_KT, _D), lambda i: (i, 0)),
        ],
        out_shape=[
            jax.ShapeDtypeStruct((_B * _N, _D), jnp.float32),
            jax.ShapeDtypeStruct((_K, _D), jnp.float32),
        ],
    )(x, x_means)

    buckets = pl.pallas_call(
        _assign_body,
        grid=(_NT, _NK),
        in_specs=[
            pl.BlockSpec((_TT, _D), lambda i, j: (i, 0)),
            pl.BlockSpec((_KT, _D), lambda i, j: (j, 0)),
        ],
        out_specs=pl.BlockSpec((1, 1, _TT), lambda i, j: (i, 0, 0)),
        out_shape=jax.ShapeDtypeStruct((_NT, 1, _TT), jnp.int32),
        scratch_shapes=[
            pltpu.VMEM((1, _TT), jnp.float32),
            pltpu.VMEM((1, _TT), jnp.int32),
        ],
        compiler_params=pltpu.CompilerParams(
            dimension_semantics=("arbitrary", "arbitrary")),
    )(xn, mn)

    bkt_flat = buckets.reshape(_B * _N)

    sc_scatter = pl.kernel(
        _sc_scatter_body,
        mesh=plsc.VectorSubcoreMesh(core_axis_name="c", subcore_axis_name="s"),
        out_type=[
            jax.ShapeDtypeStruct((_K, _D), jnp.float32),
            jax.ShapeDtypeStruct((_K, _CW), jnp.float32),
        ],
        scratch_types=[
            pltpu.VMEM((_TCHUNK, _D), jnp.float32),
            pltpu.VMEM((_TCHUNK,), jnp.int32),
            pltpu.VMEM((_CPW, _D), jnp.float32),
            pltpu.VMEM((_CPW, _CW), jnp.float32),
        ],
    )
    sums, cnt = sc_scatter(xn, bkt_flat)

    xg, k, v = pl.pallas_call(
        _finish_body,
        grid=(_NK,),
        in_specs=[
            pl.BlockSpec((_KT, _D), lambda j: (j, 0)),
            pl.BlockSpec((_KT, _CW), lambda j: (j, 0)),
            pl.BlockSpec((_KT, _D), lambda j: (j, 0)),
            pl.BlockSpec((_QK_DIM, _D), lambda j: (0, 0)),
            pl.BlockSpec((_D, _D), lambda j: (0, 0)),
        ],
        out_specs=[
            pl.BlockSpec((_KT, _D), lambda j: (j, 0)),
            pl.BlockSpec((_KT, _QK_DIM), lambda j: (j, 0)),
            pl.BlockSpec((_KT, _D), lambda j: (j, 0)),
        ],
        out_shape=[
            jax.ShapeDtypeStruct((_K, _D), jnp.float32),
            jax.ShapeDtypeStruct((_K, _QK_DIM), jnp.float32),
            jax.ShapeDtypeStruct((_K, _D), jnp.float32),
        ],
    )(sums, cnt, mn, W_k, W_v)

    k = k.reshape(_K, _HEADS, _QK_DIM // _HEADS).transpose(1, 0, 2)
    v = v.reshape(_K, _HEADS, _D // _HEADS).transpose(1, 0, 2)
    return (k, v, jax.lax.stop_gradient(xg))
